# baseline (speedup 1.0000x reference)
import functools

import jax
import jax.numpy as jnp
from jax import lax
from jax.experimental import pallas as pl
from jax.experimental.pallas import tpu as pltpu
from jax.experimental.pallas import tpu_sc as plsc

EPS = 1e-5


@functools.partial(jax.jit, static_argnums=(2, 3, 4, 5))
def _sc_gather(wte, x2d, ntok, d, tok_off, t_period):
    NC, NS = 2, 16
    NW = NC * NS
    b_per_w = ntok // NW
    NBUF = 4
    CH = b_per_w // NBUF
    nchunk = NBUF

    mesh = plsc.VectorSubcoreMesh(core_axis_name="c", subcore_axis_name="s")

    @functools.partial(
        pl.kernel,
        mesh=mesh,
        out_type=jax.ShapeDtypeStruct((ntok, d), jnp.float32),
        scratch_types=(
            [pltpu.VMEM((b_per_w,), jnp.int32)]
            + [pltpu.VMEM((CH, d), jnp.float32) for _ in range(NBUF)]
            + [pltpu.SemaphoreType.DMA for _ in range(2 * NBUF)]
        ),
    )
    def gather_kernel(table_hbm, idx_hbm, out_hbm, idx_v, *bufs):
        rows = bufs[:NBUF]
        gsems = bufs[NBUF:2 * NBUF]
        wsems = bufs[2 * NBUF:]
        wid = lax.axis_index("s") * NC + lax.axis_index("c")
        base = wid * b_per_w
        g0 = tok_off + base
        pltpu.sync_copy(
            idx_hbm.at[g0 // t_period, pl.ds(g0 % t_period, b_per_w)], idx_v)

        gcopies = [
            pltpu.async_copy(
                table_hbm.at[idx_v.at[pl.ds(c * CH, CH)]], rows[c], gsems[c])
            for c in range(nchunk)
        ]
        wcopies = []
        for c in range(nchunk):
            gcopies[c].wait()
            wcopies.append(pltpu.async_copy(
                rows[c], out_hbm.at[pl.ds(base + c * CH, CH)], wsems[c]))
        for w in wcopies:
            w.wait()

    return gather_kernel(wte, x2d)


def _tc_body(blk0, per_t, blk, tok_ref, wpe_ref, gamma_ref, beta_ref, w_ref,
             b_ref, *rest):
    out_ref = rest[-1]
    t0 = ((blk0 + pl.program_id(0)) % per_t) * blk
    y = tok_ref[...] + wpe_ref[pl.ds(t0, blk), :]
    mu = jnp.mean(y, axis=1, keepdims=True)
    yc = y - mu
    var = jnp.mean(yc * yc, axis=1, keepdims=True)
    z = yc * lax.rsqrt(var + EPS) * gamma_ref[...] + beta_ref[...]
    out_ref[...] = (
        lax.dot_general(z, w_ref[...], (((1,), (1,)), ((), ())),
                        preferred_element_type=jnp.float32)
        + b_ref[...]
    )


def _tc_chunk_call(prev, tokens_c, wpe, gamma, beta, W, b, t_period, tok0,
                   ntok_total):
    ntok_c, d = tokens_c.shape
    BLK = t_period
    nblk = ntok_c // BLK
    per_t = t_period // BLK
    blk0 = tok0 // BLK
    has_prev = prev is not None

    in_specs = [
        pl.BlockSpec((BLK, d), lambda i: (i, 0)),
        pl.BlockSpec((t_period, d), lambda i: (0, 0)),
        pl.BlockSpec((d,), lambda i: (0,)),
        pl.BlockSpec((d,), lambda i: (0,)),
        pl.BlockSpec((d, d), lambda i: (0, 0)),
        pl.BlockSpec((d,), lambda i: (0,)),
    ]
    args = [tokens_c, wpe, gamma, beta, W, b]
    aliases = {}
    if has_prev:
        in_specs.append(pl.BlockSpec(memory_space=pltpu.MemorySpace.HBM))
        args.append(prev)
        aliases = {6: 0}

    return pl.pallas_call(
        functools.partial(_tc_body, blk0, per_t, BLK),
        grid=(nblk,),
        in_specs=in_specs,
        out_specs=pl.BlockSpec((BLK, d), lambda i, b0=blk0: (b0 + i, 0)),
        out_shape=jax.ShapeDtypeStruct((ntok_total, d), jnp.float32),
        input_output_aliases=aliases,
    )(*args)


def kernel(x, wte, wpe, gamma, beta, W, b):
    B, T = x.shape
    V, D = wte.shape
    ntok = B * T
    x = x.astype(jnp.int32)
    chunks = [(0, T), (T, T), (2 * T, 2 * T)]
    toks = [_sc_gather(wte, x, n, D, off, T) for off, n in chunks]
    out = None
    for (off, n), tok in zip(chunks, toks):
        out = _tc_chunk_call(out, tok, wpe, gamma, beta, W, b, T, off, ntok)
    return out.reshape(B, T, D)


# speedup vs baseline: 1.0982x; 1.0982x over previous
import functools

import jax
import jax.numpy as jnp
from jax import lax
from jax.experimental import pallas as pl
from jax.experimental.pallas import tpu as pltpu
from jax.experimental.pallas import tpu_sc as plsc

EPS = 1e-5


@functools.partial(jax.jit, static_argnums=(2, 3, 4, 5))
def _sc_gather(wte, x2d, d, phase, tsub, t_period):
    NC, NS = 2, 16
    NW = NC * NS
    nbatch = x2d.shape[0]
    ntok = nbatch * tsub
    b_per_w = ntok // NW
    NBUF = 4
    CH = b_per_w // NBUF
    nchunk = NBUF

    mesh = plsc.VectorSubcoreMesh(core_axis_name="c", subcore_axis_name="s")

    @functools.partial(
        pl.kernel,
        mesh=mesh,
        out_type=jax.ShapeDtypeStruct((ntok, d), jnp.float32),
        scratch_types=(
            [pltpu.VMEM((b_per_w,), jnp.int32)]
            + [pltpu.VMEM((CH, d), jnp.float32) for _ in range(NBUF)]
            + [pltpu.SemaphoreType.DMA for _ in range(2 * NBUF)]
        ),
    )
    def gather_kernel(table_hbm, idx_hbm, out_hbm, idx_v, *bufs):
        rows = bufs[:NBUF]
        gsems = bufs[NBUF:2 * NBUF]
        wsems = bufs[2 * NBUF:]
        wid = lax.axis_index("s") * NC + lax.axis_index("c")
        base = wid * b_per_w
        brow = base // tsub
        col = base % tsub + phase * tsub
        pltpu.sync_copy(idx_hbm.at[brow, pl.ds(col, b_per_w)], idx_v)

        gcopies = [
            pltpu.async_copy(
                table_hbm.at[idx_v.at[pl.ds(c * CH, CH)]], rows[c], gsems[c])
            for c in range(nchunk)
        ]
        wcopies = []
        for c in range(nchunk):
            gcopies[c].wait()
            wcopies.append(pltpu.async_copy(
                rows[c], out_hbm.at[pl.ds(base + c * CH, CH)], wsems[c]))
        for w in wcopies:
            w.wait()

    return gather_kernel(wte, x2d)


def _tc_body(tok_ref, wpe_ref, gamma_ref, beta_ref, w_ref, b_ref, *rest):
    out_ref = rest[-1]
    y = tok_ref[...] + wpe_ref[...]
    mu = jnp.mean(y, axis=1, keepdims=True)
    yc = y - mu
    var = jnp.mean(yc * yc, axis=1, keepdims=True)
    z = yc * lax.rsqrt(var + EPS) * gamma_ref[...] + beta_ref[...]
    out_ref[...] = (
        lax.dot_general(z, w_ref[...], (((1,), (1,)), ((), ())),
                        preferred_element_type=jnp.float32)
        + b_ref[...]
    )


def _tc_chunk_call(prev, tokens_c, wpe, gamma, beta, W, b, phase, nphase,
                   tsub, ntok_total):
    ntok_c, d = tokens_c.shape
    BLK = tsub
    nblk = ntok_c // BLK
    has_prev = prev is not None

    in_specs = [
        pl.BlockSpec((BLK, d), lambda i: (i, 0)),
        pl.BlockSpec((tsub, d), lambda i, p=phase: (p, 0)),
        pl.BlockSpec((d,), lambda i: (0,)),
        pl.BlockSpec((d,), lambda i: (0,)),
        pl.BlockSpec((d, d), lambda i: (0, 0)),
        pl.BlockSpec((d,), lambda i: (0,)),
    ]
    args = [tokens_c, wpe, gamma, beta, W, b]
    aliases = {}
    if has_prev:
        in_specs.append(pl.BlockSpec(memory_space=pltpu.MemorySpace.HBM))
        args.append(prev)
        aliases = {6: 0}

    return pl.pallas_call(
        _tc_body,
        grid=(nblk,),
        in_specs=in_specs,
        out_specs=pl.BlockSpec(
            (BLK, d), lambda i, p=phase, np_=nphase: (np_ * i + p, 0)),
        out_shape=jax.ShapeDtypeStruct((ntok_total, d), jnp.float32),
        input_output_aliases=aliases,
    )(*args)


def kernel(x, wte, wpe, gamma, beta, W, b):
    B, T = x.shape
    V, D = wte.shape
    ntok = B * T
    x = x.astype(jnp.int32)
    NPH = 2
    tsub = T // NPH
    toks = [_sc_gather(wte, x, D, p, tsub, T) for p in range(NPH)]
    out = None
    for p in range(NPH):
        out = _tc_chunk_call(out, toks[p], wpe, gamma, beta, W, b,
                             p, NPH, tsub, ntok)
    return out.reshape(B, T, D)


# speedup vs baseline: 1.0986x; 1.0004x over previous
import functools

import jax
import jax.numpy as jnp
from jax import lax
from jax.experimental import pallas as pl
from jax.experimental.pallas import tpu as pltpu
from jax.experimental.pallas import tpu_sc as plsc

EPS = 1e-5


@functools.partial(jax.jit, static_argnums=(2, 3, 4, 5))
def _sc_gather(wte, x2d, d, phase, tsub, t_period):
    NC, NS = 2, 16
    NW = NC * NS
    nbatch = x2d.shape[0]
    ntok = nbatch * tsub
    b_per_w = ntok // NW
    NBUF = 4
    CH = b_per_w // NBUF
    nchunk = NBUF

    mesh = plsc.VectorSubcoreMesh(core_axis_name="c", subcore_axis_name="s")

    @functools.partial(
        pl.kernel,
        mesh=mesh,
        out_type=jax.ShapeDtypeStruct((ntok, d), jnp.float32),
        scratch_types=(
            [pltpu.VMEM((b_per_w,), jnp.int32)]
            + [pltpu.VMEM((CH, d), jnp.float32) for _ in range(NBUF)]
            + [pltpu.SemaphoreType.DMA for _ in range(2 * NBUF)]
        ),
    )
    def gather_kernel(table_hbm, idx_hbm, out_hbm, idx_v, *bufs):
        rows = bufs[:NBUF]
        gsems = bufs[NBUF:2 * NBUF]
        wsems = bufs[2 * NBUF:]
        wid = lax.axis_index("s") * NC + lax.axis_index("c")
        base = wid * b_per_w
        brow = base // tsub
        col = base % tsub + phase * tsub
        pltpu.sync_copy(idx_hbm.at[brow, pl.ds(col, b_per_w)], idx_v)

        gcopies = [
            pltpu.async_copy(
                table_hbm.at[idx_v.at[pl.ds(c * CH, CH)]], rows[c], gsems[c])
            for c in range(nchunk)
        ]
        wcopies = []
        for c in range(nchunk):
            gcopies[c].wait()
            wcopies.append(pltpu.async_copy(
                rows[c], out_hbm.at[pl.ds(base + c * CH, CH)], wsems[c]))
        for w in wcopies:
            w.wait()

    return gather_kernel(wte, x2d)


def _tc_body(tok_ref, wpe_ref, gamma_ref, beta_ref, w_ref, b_ref, *rest):
    out_ref = rest[-1]
    y = tok_ref[...] + wpe_ref[...]
    mu = jnp.mean(y, axis=1, keepdims=True)
    m2 = jnp.mean(y * y, axis=1, keepdims=True)
    rs = lax.rsqrt(m2 - mu * mu + EPS)
    z = (y - mu) * (rs * gamma_ref[...]) + beta_ref[...]
    out_ref[...] = (
        lax.dot_general(z, w_ref[...], (((1,), (1,)), ((), ())),
                        preferred_element_type=jnp.float32)
        + b_ref[...]
    )


def _tc_chunk_call(prev, tokens_c, wpe, gamma, beta, W, b, phase, nphase,
                   tsub, ntok_total):
    ntok_c, d = tokens_c.shape
    BLK = tsub
    nblk = ntok_c // BLK
    has_prev = prev is not None

    in_specs = [
        pl.BlockSpec((BLK, d), lambda i: (i, 0)),
        pl.BlockSpec((tsub, d), lambda i, p=phase: (p, 0)),
        pl.BlockSpec((d,), lambda i: (0,)),
        pl.BlockSpec((d,), lambda i: (0,)),
        pl.BlockSpec((d, d), lambda i: (0, 0)),
        pl.BlockSpec((d,), lambda i: (0,)),
    ]
    args = [tokens_c, wpe, gamma, beta, W, b]
    aliases = {}
    if has_prev:
        in_specs.append(pl.BlockSpec(memory_space=pltpu.MemorySpace.HBM))
        args.append(prev)
        aliases = {6: 0}

    return pl.pallas_call(
        _tc_body,
        grid=(nblk,),
        in_specs=in_specs,
        out_specs=pl.BlockSpec(
            (BLK, d), lambda i, p=phase, np_=nphase: (np_ * i + p, 0)),
        out_shape=jax.ShapeDtypeStruct((ntok_total, d), jnp.float32),
        input_output_aliases=aliases,
    )(*args)


def kernel(x, wte, wpe, gamma, beta, W, b):
    B, T = x.shape
    V, D = wte.shape
    ntok = B * T
    x = x.astype(jnp.int32)
    NPH = 2
    tsub = T // NPH
    toks = [_sc_gather(wte, x, D, p, tsub, T) for p in range(NPH)]
    out = None
    for p in range(NPH):
        out = _tc_chunk_call(out, toks[p], wpe, gamma, beta, W, b,
                             p, NPH, tsub, ntok)
    return out.reshape(B, T, D)
